# baseline (speedup 1.0000x reference)
import functools

import jax
import jax.numpy as jnp
from jax import lax
from jax.experimental import pallas as pl
from jax.experimental.pallas import tpu as pltpu
from jax.experimental.pallas import tpu_sc as plsc

VOCAB_SIZE = 100000
D_MODEL = 1024
MAX_LEN = 8192
BATCH = 4
SEQ_LEN = 4096

NUM_CORES = 2
NUM_SUBCORES = 16
NUM_WORKERS = NUM_CORES * NUM_SUBCORES
N_ROWS = BATCH * SEQ_LEN
ROWS_PER_WORKER = N_ROWS // NUM_WORKERS
CHUNK = 16
N_CHUNKS = ROWS_PER_WORKER // CHUNK
LANES = 16
GROUPS = D_MODEL // LANES


def _body(x_hbm, tok_hbm, pos_hbm, out_hbm, idx_v, tok_v, pos_v, sem_g, sem_p):
    wid = lax.axis_index("s") * NUM_CORES + lax.axis_index("c")
    base = wid * ROWS_PER_WORKER
    pos_base = lax.rem(base, SEQ_LEN)

    pltpu.sync_copy(x_hbm.at[pl.ds(base, ROWS_PER_WORKER)], idx_v)

    def chunk_step(c, _):
        off = c * CHUNK
        g = pltpu.async_copy(tok_hbm.at[idx_v.at[pl.ds(off, CHUNK)]], tok_v, sem_g)
        p = pltpu.async_copy(pos_hbm.at[pl.ds(pos_base + off, CHUNK)], pos_v, sem_p)
        g.wait()
        p.wait()

        def row_add(i, _):
            for grp in range(GROUPS):
                sl = pl.ds(grp * LANES, LANES)
                tok_v[i, sl] = tok_v[i, sl] + pos_v[i, sl]
            return 0

        lax.fori_loop(0, CHUNK, row_add, 0, unroll=False)
        pltpu.sync_copy(tok_v, out_hbm.at[pl.ds(base + off, CHUNK)])
        return 0

    lax.fori_loop(0, N_CHUNKS, chunk_step, 0, unroll=False)


@jax.jit
def _run(x_flat, token_table, position_table):
    mesh = plsc.VectorSubcoreMesh(core_axis_name="c", subcore_axis_name="s")
    k = pl.kernel(
        _body,
        out_type=jax.ShapeDtypeStruct((N_ROWS, D_MODEL), jnp.float32),
        mesh=mesh,
        scratch_types=[
            pltpu.VMEM((ROWS_PER_WORKER,), jnp.int32),
            pltpu.VMEM((CHUNK, D_MODEL), jnp.float32),
            pltpu.VMEM((CHUNK, D_MODEL), jnp.float32),
            pltpu.SemaphoreType.DMA,
            pltpu.SemaphoreType.DMA,
        ],
    )
    return k(x_flat, token_table, position_table)


def kernel(x, token_table, position_table):
    x_flat = x.reshape(N_ROWS).astype(jnp.int32)
    out = _run(x_flat, token_table, position_table)
    return out.reshape(BATCH, SEQ_LEN, D_MODEL)


# speedup vs baseline: 1.5720x; 1.5720x over previous
import functools

import jax
import jax.numpy as jnp
from jax import lax
from jax.experimental import pallas as pl
from jax.experimental.pallas import tpu as pltpu
from jax.experimental.pallas import tpu_sc as plsc

VOCAB_SIZE = 100000
D_MODEL = 1024
MAX_LEN = 8192
BATCH = 4
SEQ_LEN = 4096

NUM_CORES = 2
NUM_SUBCORES = 16
NUM_WORKERS = NUM_CORES * NUM_SUBCORES
N_ROWS = BATCH * SEQ_LEN
ROWS_PER_WORKER = N_ROWS // NUM_WORKERS
CHUNK = 16
N_CHUNKS = ROWS_PER_WORKER // CHUNK
LANES = 16
GROUPS = D_MODEL // LANES


def _body(x_hbm, tok_hbm, pos_hbm, out_hbm, idx_v,
          tok0, tok1, pos0, pos1, sg0, sg1, sp0, sp1, so0, so1):
    wid = lax.axis_index("s") * NUM_CORES + lax.axis_index("c")
    base = wid * ROWS_PER_WORKER
    pos_base = lax.rem(base, SEQ_LEN)

    toks = (tok0, tok1)
    poss = (pos0, pos1)
    sgs = (sg0, sg1)
    sps = (sp0, sp1)
    sos = (so0, so1)

    pltpu.sync_copy(x_hbm.at[pl.ds(base, ROWS_PER_WORKER)], idx_v)

    def start_chunk(c, b):
        off = c * CHUNK
        pltpu.async_copy(tok_hbm.at[idx_v.at[pl.ds(off, CHUNK)]], toks[b], sgs[b])
        pltpu.async_copy(pos_hbm.at[pl.ds(pos_base + off, CHUNK)], poss[b], sps[b])

    def wait_chunk(b):
        pltpu.make_async_copy(tok_hbm.at[pl.ds(0, CHUNK)], toks[b], sgs[b]).wait()
        pltpu.make_async_copy(pos_hbm.at[pl.ds(0, CHUNK)], poss[b], sps[b]).wait()

    def start_out(c, b):
        pltpu.async_copy(poss[b], out_hbm.at[pl.ds(base + c * CHUNK, CHUNK)], sos[b])

    def wait_out(b):
        pltpu.make_async_copy(poss[b], out_hbm.at[pl.ds(base, CHUNK)], sos[b]).wait()

    def add_chunk(b):
        tok_v, pos_v = toks[b], poss[b]

        def row_add(i, _):
            for grp in range(GROUPS):
                sl = pl.ds(grp * LANES, LANES)
                pos_v[i, sl] = tok_v[i, sl] + pos_v[i, sl]
            return 0

        lax.fori_loop(0, CHUNK, row_add, 0, unroll=False)

    start_chunk(0, 0)

    @pl.loop(0, N_CHUNKS, step=2)
    def _chunk_pair(i):
        for b in (0, 1):
            c = i + b
            nb = 1 - b
            if b == 0:
                @pl.when(i > 0)
                def _():
                    wait_out(nb)

                start_chunk(c + 1, nb)
            else:
                @pl.when(i < N_CHUNKS - 2)
                def _():
                    wait_out(nb)
                    start_chunk(c + 1, nb)

            wait_chunk(b)
            add_chunk(b)
            start_out(c, b)

    wait_out(0)
    wait_out(1)


@jax.jit
def _run(x_flat, token_table, position_table):
    mesh = plsc.VectorSubcoreMesh(core_axis_name="c", subcore_axis_name="s")
    k = pl.kernel(
        _body,
        out_type=jax.ShapeDtypeStruct((N_ROWS, D_MODEL), jnp.float32),
        mesh=mesh,
        scratch_types=[
            pltpu.VMEM((ROWS_PER_WORKER,), jnp.int32),
            pltpu.VMEM((CHUNK, D_MODEL), jnp.float32),
            pltpu.VMEM((CHUNK, D_MODEL), jnp.float32),
            pltpu.VMEM((CHUNK, D_MODEL), jnp.float32),
            pltpu.VMEM((CHUNK, D_MODEL), jnp.float32),
            pltpu.SemaphoreType.DMA,
            pltpu.SemaphoreType.DMA,
            pltpu.SemaphoreType.DMA,
            pltpu.SemaphoreType.DMA,
            pltpu.SemaphoreType.DMA,
            pltpu.SemaphoreType.DMA,
        ],
    )
    return k(x_flat, token_table, position_table)


def kernel(x, token_table, position_table):
    x_flat = x.reshape(N_ROWS).astype(jnp.int32)
    out = _run(x_flat, token_table, position_table)
    return out.reshape(BATCH, SEQ_LEN, D_MODEL)


# speedup vs baseline: 1.5827x; 1.0068x over previous
import functools

import jax
import jax.numpy as jnp
from jax import lax
from jax.experimental import pallas as pl
from jax.experimental.pallas import tpu as pltpu
from jax.experimental.pallas import tpu_sc as plsc

VOCAB_SIZE = 100000
D_MODEL = 1024
MAX_LEN = 8192
BATCH = 4
SEQ_LEN = 4096

NUM_CORES = 2
NUM_SUBCORES = 16
NUM_WORKERS = NUM_CORES * NUM_SUBCORES
N_ROWS = BATCH * SEQ_LEN
S_BLOCK = SEQ_LEN // NUM_WORKERS
CHUNK = 16
N_PCHUNKS = S_BLOCK // CHUNK
LANES = 16
GROUPS = D_MODEL // LANES
LAST_I = N_PCHUNKS - 2


def _body(x_hbm, tok_hbm, pos_hbm, out_hbm, idx_v,
          tok0, tok1, pos0, pos1, sg0, sg1, sp0, sp1, so0, so1):
    wid = lax.axis_index("s") * NUM_CORES + lax.axis_index("c")
    s_base = wid * S_BLOCK

    toks = (tok0, tok1)
    poss = (pos0, pos1)
    sgs = (sg0, sg1)
    sos = (so0, so1)

    pltpu.sync_copy(x_hbm.at[pl.ds(wid * BATCH * S_BLOCK, BATCH * S_BLOCK)], idx_v)

    def start_gather(c, b, tb):
        pltpu.async_copy(
            tok_hbm.at[idx_v.at[pl.ds(b * S_BLOCK + c * CHUNK, CHUNK)]],
            toks[tb], sgs[tb])

    def wait_gather(tb):
        pltpu.make_async_copy(tok_hbm.at[pl.ds(0, CHUNK)], toks[tb], sgs[tb]).wait()

    sps = (sp0, sp1)

    def start_pos(c, pb):
        pltpu.async_copy(pos_hbm.at[pl.ds(s_base + c * CHUNK, CHUNK)],
                         poss[pb], sps[pb])

    def wait_pos(pb):
        pltpu.make_async_copy(pos_hbm.at[pl.ds(0, CHUNK)], poss[pb], sps[pb]).wait()

    def start_out(c, b, tb):
        pltpu.async_copy(
            toks[tb],
            out_hbm.at[pl.ds(b * SEQ_LEN + s_base + c * CHUNK, CHUNK)],
            sos[tb])

    def wait_out(tb):
        pltpu.make_async_copy(toks[tb], out_hbm.at[pl.ds(0, CHUNK)], sos[tb]).wait()

    def add_chunk(tb, pb):
        tok_v, pos_v = toks[tb], poss[pb]

        def row_add(i, _):
            for grp in range(GROUPS):
                sl = pl.ds(grp * LANES, LANES)
                tok_v[i, sl] = tok_v[i, sl] + pos_v[i, sl]
            return 0

        lax.fori_loop(0, CHUNK, row_add, 0, unroll=False)

    start_pos(0, 0)
    start_gather(0, 0, 0)

    @pl.loop(0, N_PCHUNKS, step=2)
    def _chunk_pair(i):
        for cc in (0, 1):
            c = i + cc
            pb = cc
            if cc == 0:
                start_pos(c + 1, 1 - pb)
            else:
                @pl.when(i < LAST_I)
                def _():
                    start_pos(c + 1, 1 - pb)

            wait_pos(pb)

            for b in range(BATCH):
                tb = b % 2
                nt = 1 - tb
                if cc == 0 and b == 0:
                    @pl.when(i > 0)
                    def _():
                        wait_out(nt)
                elif cc == 1 and b == BATCH - 1:
                    @pl.when(i < LAST_I)
                    def _():
                        wait_out(nt)
                else:
                    wait_out(nt)

                if b < BATCH - 1:
                    start_gather(c, b + 1, nt)
                elif cc == 0:
                    start_gather(c + 1, 0, nt)
                else:
                    @pl.when(i < LAST_I)
                    def _():
                        start_gather(c + 1, 0, nt)

                wait_gather(tb)
                add_chunk(tb, pb)
                start_out(c, b, tb)

    wait_out(0)
    wait_out(1)


@jax.jit
def _run(x_flat, token_table, position_table):
    mesh = plsc.VectorSubcoreMesh(core_axis_name="c", subcore_axis_name="s")
    k = pl.kernel(
        _body,
        out_type=jax.ShapeDtypeStruct((N_ROWS, D_MODEL), jnp.float32),
        mesh=mesh,
        scratch_types=[
            pltpu.VMEM((BATCH * S_BLOCK,), jnp.int32),
            pltpu.VMEM((CHUNK, D_MODEL), jnp.float32),
            pltpu.VMEM((CHUNK, D_MODEL), jnp.float32),
            pltpu.VMEM((CHUNK, D_MODEL), jnp.float32),
            pltpu.VMEM((CHUNK, D_MODEL), jnp.float32),
            pltpu.SemaphoreType.DMA,
            pltpu.SemaphoreType.DMA,
            pltpu.SemaphoreType.DMA,
            pltpu.SemaphoreType.DMA,
            pltpu.SemaphoreType.DMA,
            pltpu.SemaphoreType.DMA,
        ],
    )
    return k(x_flat, token_table, position_table)


def kernel(x, token_table, position_table):
    x_flat = (x.astype(jnp.int32)
              .reshape(BATCH, NUM_WORKERS, S_BLOCK)
              .transpose(1, 0, 2)
              .reshape(N_ROWS))
    out = _run(x_flat, token_table, position_table)
    return out.reshape(BATCH, SEQ_LEN, D_MODEL)
